# baseline (speedup 1.0000x reference)
import functools

import jax
import jax.numpy as jnp
from jax import lax
from jax.experimental import pallas as pl
from jax.experimental.pallas import tpu as pltpu
from jax.experimental.pallas import tpu_sc as plsc

_GATHER_DNUMS = lax.GatherDimensionNumbers(
    offset_dims=(), collapsed_slice_dims=(0,), start_index_map=(0,))


def _lane_shuffle(v, perm):
    return lax.gather(v, perm[:, None], _GATHER_DNUMS, slice_sizes=(1,),
                      mode=lax.GatherScatterMode.PROMISE_IN_BOUNDS)


_N_ENT = 1000000
_N_REL = 1000
_D = 64
_B = 16384
_L = 16
_NW = 32
_BPW = _B // _NW
_C = 128
_NCHUNK = _BPW // _C


def _make_sc_kernel():
    mesh = plsc.VectorSubcoreMesh(core_axis_name="c", subcore_axis_name="s")

    @functools.partial(
        pl.kernel,
        mesh=mesh,
        out_type=jax.ShapeDtypeStruct((_B,), jnp.float32),
        scratch_types=[
            pltpu.VMEM((_C,), jnp.int32),
            pltpu.VMEM((_C,), jnp.int32),
            pltpu.VMEM((_C,), jnp.int32),
            pltpu.VMEM((_C, _D), jnp.float32),
            pltpu.VMEM((_C, _D), jnp.float32),
            pltpu.VMEM((_C, _D), jnp.float32),
            pltpu.VMEM((_C, _D), jnp.float32),
            pltpu.VMEM((_C, _D), jnp.float32),
            pltpu.VMEM((_C, _D), jnp.float32),
            pltpu.VMEM((_BPW,), jnp.float32),
            pltpu.SemaphoreType.DMA,
        ],
        compiler_params=pltpu.CompilerParams(use_tc_tiling_on_sc=False),
    )
    def complex_score(heads_hbm, rels_hbm, tails_hbm,
                      er_hbm, ei_hbm, rr_hbm, ri_hbm,
                      out_hbm,
                      idx_h, idx_t, idx_r,
                      hr_v, hi_v, tr_v, ti_v, rr_v, ri_v,
                      out_v, sem):
        wid = lax.axis_index("s") * 2 + lax.axis_index("c")
        base = pl.multiple_of(wid * _BPW, _BPW)
        lanes = lax.iota(jnp.int32, _L)
        perms = [lanes ^ sh for sh in (8, 4, 2, 1)]
        lane_masks = [lanes == k for k in range(_L)]

        for c in range(_NCHUNK):
            cbase = base + c * _C
            pltpu.sync_copy(heads_hbm.at[pl.ds(cbase, _C)], idx_h)
            pltpu.sync_copy(tails_hbm.at[pl.ds(cbase, _C)], idx_t)
            pltpu.sync_copy(rels_hbm.at[pl.ds(cbase, _C)], idx_r)

            cps = [
                pltpu.async_copy(er_hbm.at[idx_h], hr_v, sem),
                pltpu.async_copy(ei_hbm.at[idx_h], hi_v, sem),
                pltpu.async_copy(er_hbm.at[idx_t], tr_v, sem),
                pltpu.async_copy(ei_hbm.at[idx_t], ti_v, sem),
                pltpu.async_copy(rr_hbm.at[idx_r], rr_v, sem),
                pltpu.async_copy(ri_hbm.at[idx_r], ri_v, sem),
            ]
            for cp in cps:
                cp.wait()

            def group_body(g, carry, _c=c):
                out_vec = jnp.zeros((_L,), jnp.float32)
                for k in range(_L):
                    i = g * _L + k
                    acc = jnp.zeros((_L,), jnp.float32)
                    for j in range(_D // _L):
                        sl = pl.ds(j * _L, _L)
                        hr = hr_v[i, sl]
                        hi = hi_v[i, sl]
                        tr = tr_v[i, sl]
                        ti = ti_v[i, sl]
                        s = rr_v[i, sl] + ri_v[i, sl]
                        acc = acc + s * ((hr - hi) * tr + (hr + hi) * ti)
                    for perm in perms:
                        acc = acc + _lane_shuffle(acc, perm)
                    out_vec = lax.select(lane_masks[k], acc, out_vec)
                out_v[pl.ds(_c * _C + g * _L, _L)] = out_vec
                return carry

            lax.fori_loop(0, _C // _L, group_body, 0)

        pltpu.sync_copy(out_v, out_hbm.at[pl.ds(base, _BPW)])

    return complex_score


_sc_kernel = _make_sc_kernel()


def kernel(heads, relations, tails, entity_real, entity_imag,
           relation_real, relation_imag):
    return _sc_kernel(heads.astype(jnp.int32), relations.astype(jnp.int32),
                      tails.astype(jnp.int32), entity_real, entity_imag,
                      relation_real, relation_imag)


# speedup vs baseline: 1.5416x; 1.5416x over previous
import functools

import jax
import jax.numpy as jnp
from jax import lax
from jax.experimental import pallas as pl
from jax.experimental.pallas import tpu as pltpu
from jax.experimental.pallas import tpu_sc as plsc

_D = 64
_B = 16384
_L = 16
_NW = 32
_BPW = _B // _NW
_C = 128
_NCHUNK = _BPW // _C
_NG = _C // _L


def _make_sc_kernel():
    mesh = plsc.VectorSubcoreMesh(core_axis_name="c", subcore_axis_name="s")

    @functools.partial(
        pl.kernel,
        mesh=mesh,
        out_type=jax.ShapeDtypeStruct((_B,), jnp.float32),
        scratch_types=[
            pltpu.VMEM((_BPW,), jnp.int32),
            pltpu.VMEM((_BPW,), jnp.int32),
            pltpu.VMEM((_BPW,), jnp.int32),
            pltpu.VMEM((_C, _D), jnp.float32),
            pltpu.VMEM((_C, _D), jnp.float32),
            pltpu.VMEM((_C, _D), jnp.float32),
            pltpu.VMEM((_C, _D), jnp.float32),
            pltpu.VMEM((_C, _D), jnp.float32),
            pltpu.VMEM((_C, _D), jnp.float32),
            pltpu.VMEM((_BPW,), jnp.float32),
            pltpu.SemaphoreType.DMA,
        ],
        compiler_params=pltpu.CompilerParams(use_tc_tiling_on_sc=True),
    )
    def complex_score(heads_hbm, rels_hbm, tails_hbm,
                      er_hbm, ei_hbm, rr_hbm, ri_hbm,
                      out_hbm,
                      idx_h, idx_t, idx_r,
                      hr_v, hi_v, tr_v, ti_v, rr_v, ri_v,
                      out_v, sem):
        wid = lax.axis_index("s") * 2 + lax.axis_index("c")
        base = pl.multiple_of(wid * _BPW, _BPW)
        lanes = lax.iota(jnp.int32, _L)
        perms = [lanes ^ sh for sh in (8, 4, 2, 1)]
        lane_masks = [lanes == k for k in range(_L)]

        pltpu.sync_copy(heads_hbm.at[pl.ds(base, _BPW)], idx_h)
        pltpu.sync_copy(tails_hbm.at[pl.ds(base, _BPW)], idx_t)
        pltpu.sync_copy(rels_hbm.at[pl.ds(base, _BPW)], idx_r)

        def chunk_body(c, carry):
            def issue_body(g, carry2):
                hvec = idx_h[pl.ds(c * _C + g * _L, _L)]
                tvec = idx_t[pl.ds(c * _C + g * _L, _L)]
                rvec = idx_r[pl.ds(c * _C + g * _L, _L)]
                for k in range(_L):
                    i = g * _L + k
                    pltpu.async_copy(er_hbm.at[hvec[k]], hr_v.at[i], sem)
                    pltpu.async_copy(ei_hbm.at[hvec[k]], hi_v.at[i], sem)
                    pltpu.async_copy(er_hbm.at[tvec[k]], tr_v.at[i], sem)
                    pltpu.async_copy(ei_hbm.at[tvec[k]], ti_v.at[i], sem)
                    pltpu.async_copy(rr_hbm.at[rvec[k]], rr_v.at[i], sem)
                    pltpu.async_copy(ri_hbm.at[rvec[k]], ri_v.at[i], sem)
                return carry2

            lax.fori_loop(0, _NG, issue_body, 0)

            for buf in (hr_v, hi_v, tr_v, ti_v, rr_v, ri_v):
                pltpu.make_async_copy(er_hbm.at[pl.ds(0, _C)], buf, sem).wait()

            def group_body(g, carry2):
                out_vec = jnp.zeros((_L,), jnp.float32)
                for k in range(_L):
                    i = g * _L + k
                    acc = jnp.zeros((_L,), jnp.float32)
                    for j in range(_D // _L):
                        sl = pl.ds(j * _L, _L)
                        hr = hr_v[i, sl]
                        hi = hi_v[i, sl]
                        tr = tr_v[i, sl]
                        ti = ti_v[i, sl]
                        s = rr_v[i, sl] + ri_v[i, sl]
                        acc = acc + s * ((hr - hi) * tr + (hr + hi) * ti)
                    for perm in perms:
                        acc = acc + _lane_shuffle(acc, perm)
                    out_vec = lax.select(lane_masks[k], acc, out_vec)
                out_v[pl.ds(c * _C + g * _L, _L)] = out_vec
                return carry2

            lax.fori_loop(0, _NG, group_body, 0)
            return carry

        lax.fori_loop(0, _NCHUNK, chunk_body, 0)
        pltpu.sync_copy(out_v, out_hbm.at[pl.ds(base, _BPW)])

    return complex_score


_GATHER_DNUMS = lax.GatherDimensionNumbers(
    offset_dims=(), collapsed_slice_dims=(0,), start_index_map=(0,))


def _lane_shuffle(v, perm):
    return lax.gather(v, perm[:, None], _GATHER_DNUMS, slice_sizes=(1,),
                      mode=lax.GatherScatterMode.PROMISE_IN_BOUNDS)


_sc_kernel = _make_sc_kernel()


def kernel(heads, relations, tails, entity_real, entity_imag,
           relation_real, relation_imag):
    return _sc_kernel(heads.astype(jnp.int32), relations.astype(jnp.int32),
                      tails.astype(jnp.int32), entity_real, entity_imag,
                      relation_real, relation_imag)
